# baseline (speedup 1.0000x reference)
import functools

import jax
import jax.numpy as jnp
from jax import lax
from jax.experimental import pallas as pl
from jax.experimental.pallas import tpu as pltpu
from jax.experimental.pallas import tpu_sc as plsc

N = 10000
IN = 128
H = 8
C = 16
HC = H * C
E = 320000
EP = E + N

K = 80
NW = 32
CH = 130
E_PAD = NW * K * CH
PER_W = CH * K

R_ROWS = 10240
ROWS_PER_TILE = R_ROWS // 16

_BN = 2000


def _tc_prep_body(x_ref, w_ref, w2_ref, asf_ref, adf_ref, q_ref,
                  xpb_ref, as_ref, ad_ref):
    xp = jnp.dot(x_ref[...], w_ref[...], preferred_element_type=jnp.float32,
                 precision=lax.Precision.HIGHEST)
    xpb_ref[...] = jnp.dot(x_ref[...], w2_ref[...],
                           preferred_element_type=jnp.float32,
                           precision=lax.Precision.HIGHEST).astype(jnp.bfloat16)
    as_ref[...] = jnp.dot(xp * asf_ref[...], q_ref[...],
                          preferred_element_type=jnp.float32,
                          precision=lax.Precision.HIGHEST)
    ad_ref[...] = jnp.dot(xp * adf_ref[...], q_ref[...],
                          preferred_element_type=jnp.float32,
                          precision=lax.Precision.HIGHEST)


def _tc_prep(x, W, W2, att_src_flat, att_dst_flat, Q):
    grid = (N // _BN,)
    return pl.pallas_call(
        _tc_prep_body,
        grid=grid,
        in_specs=[
            pl.BlockSpec((_BN, IN), lambda i: (i, 0)),
            pl.BlockSpec((IN, HC), lambda i: (0, 0)),
            pl.BlockSpec((IN, HC), lambda i: (0, 0)),
            pl.BlockSpec((1, HC), lambda i: (0, 0)),
            pl.BlockSpec((1, HC), lambda i: (0, 0)),
            pl.BlockSpec((HC, 16), lambda i: (0, 0)),
        ],
        out_specs=[
            pl.BlockSpec((_BN, HC), lambda i: (i, 0)),
            pl.BlockSpec((_BN, 16), lambda i: (i, 0)),
            pl.BlockSpec((_BN, 16), lambda i: (i, 0)),
        ],
        out_shape=[
            jax.ShapeDtypeStruct((N, HC), jnp.bfloat16),
            jax.ShapeDtypeStruct((N, 16), jnp.float32),
            jax.ShapeDtypeStruct((N, 16), jnp.float32),
        ],
    )(x, W, W2, att_src_flat, att_dst_flat, Q)


def _splat(v, h):
    idx = jnp.full((16, 1), h, dtype=jnp.int32)
    dn = lax.GatherDimensionNumbers(offset_dims=(), collapsed_slice_dims=(0,),
                                    start_index_map=(0,))
    return lax.gather(v, idx, dn, slice_sizes=(1,),
                      mode=lax.GatherScatterMode.PROMISE_IN_BOUNDS)


def _sc_body(xp_h, as_h, ad_h, src_h, dst_h, z128_h, z16_h,
             s_out, d_out,
             s_sh, d_sh, srcv, dstv, dsts, rows, msg, asv, adv, wv,
             isem_s, isem_d, gsem_r, gsem_a, gsem_b, ssem_s, ssem_d):
    cid = lax.axis_index("c")
    sid = lax.axis_index("s")
    wid = sid * 2 + cid

    pltpu.sync_copy(z128_h, s_sh.at[pl.ds(sid * ROWS_PER_TILE, ROWS_PER_TILE)])
    pltpu.sync_copy(z16_h, d_sh.at[pl.ds(sid * ROWS_PER_TILE, ROWS_PER_TILE)])
    plsc.subcore_barrier()

    base0 = wid * PER_W

    def idx_issue(ch, b):
        base = base0 + ch * K
        pltpu.async_copy(src_h.at[pl.ds(base, K)], srcv.at[b], isem_s.at[b])
        pltpu.async_copy(dst_h.at[pl.ds(base, K)], dstv.at[b], isem_d.at[b])

    def idx_wait(b):
        pltpu.make_async_copy(src_h.at[pl.ds(0, K)], srcv.at[b],
                              isem_s.at[b]).wait()
        pltpu.make_async_copy(dst_h.at[pl.ds(0, K)], dstv.at[b],
                              isem_d.at[b]).wait()

    def gather_issue(b):
        pltpu.async_copy(xp_h.at[srcv.at[b]], rows.at[b], gsem_r.at[b])
        pltpu.async_copy(as_h.at[srcv.at[b]], asv.at[b], gsem_a.at[b])
        pltpu.async_copy(ad_h.at[dstv.at[b]], adv.at[b], gsem_b.at[b])

    def gather_wait(b):
        pltpu.make_async_copy(xp_h.at[srcv.at[b]], rows.at[b],
                              gsem_r.at[b]).wait()
        pltpu.make_async_copy(as_h.at[srcv.at[b]], asv.at[b],
                              gsem_a.at[b]).wait()
        pltpu.make_async_copy(ad_h.at[dstv.at[b]], adv.at[b],
                              gsem_b.at[b]).wait()

    def scatter_issue(b):
        pltpu.async_copy(wv.at[b], d_sh.at[dsts.at[b]], ssem_d.at[b],
                         add=True)
        pltpu.async_copy(msg.at[b], s_sh.at[dsts.at[b]], ssem_s.at[b],
                         add=True)

    def scatter_wait(b):
        pltpu.make_async_copy(wv.at[b], d_sh.at[dsts.at[b]],
                              ssem_d.at[b]).wait()
        pltpu.make_async_copy(msg.at[b], s_sh.at[dsts.at[b]],
                              ssem_s.at[b]).wait()

    idx_issue(0, 0)
    idx_wait(0)
    gather_issue(0)
    idx_issue(1, 1)

    @pl.loop(0, CH, step=2)
    def _chunk(ch0):
        for b in (0, 1):
            ch = ch0 + b
            gather_wait(b)

            @pl.when(ch + 2 < CH)
            def _():
                idx_issue(ch + 2, b)

            @pl.when(ch + 1 < CH)
            def _():
                @pl.when(ch >= 1)
                def _():
                    scatter_wait(1 - b)
                idx_wait(1 - b)
                gather_issue(1 - b)

            for j in range(K // 16):
                dsts.at[b][pl.ds(16 * j, 16)] = dstv.at[b][pl.ds(16 * j, 16)]

            @plsc.parallel_loop(0, K, unroll=2)
            def _edge(k):
                a = asv.at[b][k] + adv.at[b][k]
                w = jnp.exp(jnp.maximum(a, 0.2 * a))
                wv.at[b][k] = w
                for j in range(4):
                    r32 = rows.at[b][k, pl.ds(32 * j, 32)]
                    ev, od = plsc.unpack(r32, format=plsc.PackFormat.INTERLEAVED)
                    msg.at[b][k, pl.ds(16 * (2 * j), 16)] = ev * _splat(w, 2 * j)
                    msg.at[b][k, pl.ds(16 * (2 * j + 1), 16)] = (
                        od * _splat(w, 2 * j + 1))

            scatter_issue(b)

    scatter_wait(0)
    scatter_wait(1)

    plsc.subcore_barrier()
    sl = pl.ds(sid * ROWS_PER_TILE, ROWS_PER_TILE)
    pltpu.sync_copy(s_sh.at[sl], s_out.at[cid].at[sl])
    pltpu.sync_copy(d_sh.at[sl], d_out.at[cid].at[sl])


def _sc_edge_pass(xp, asrc, adst, srcp, dstp, z128, z16):
    mesh = plsc.VectorSubcoreMesh(core_axis_name="c", subcore_axis_name="s")
    f = pl.kernel(
        _sc_body,
        compiler_params=pltpu.CompilerParams(use_tc_tiling_on_sc=False,
                                             needs_layout_passes=False),
        out_type=[
            jax.ShapeDtypeStruct((2, R_ROWS, HC), jnp.float32),
            jax.ShapeDtypeStruct((2, R_ROWS, 16), jnp.float32),
        ],
        mesh=mesh,
        scratch_types=[
            pltpu.VMEM_SHARED((R_ROWS, HC), jnp.float32),
            pltpu.VMEM_SHARED((R_ROWS, 16), jnp.float32),
            pltpu.VMEM((2, K), jnp.int32),
            pltpu.VMEM((2, K), jnp.int32),
            pltpu.VMEM((2, K), jnp.int32),
            pltpu.VMEM((2, K, HC), jnp.bfloat16),
            pltpu.VMEM((2, K, HC), jnp.float32),
            pltpu.VMEM((2, K, 16), jnp.float32),
            pltpu.VMEM((2, K, 16), jnp.float32),
            pltpu.VMEM((2, K, 16), jnp.float32),
            pltpu.SemaphoreType.DMA((2,)),
            pltpu.SemaphoreType.DMA((2,)),
            pltpu.SemaphoreType.DMA((2,)),
            pltpu.SemaphoreType.DMA((2,)),
            pltpu.SemaphoreType.DMA((2,)),
            pltpu.SemaphoreType.DMA((2,)),
            pltpu.SemaphoreType.DMA((2,)),
        ],
    )
    return f(xp, asrc, adst, srcp, dstp, z128, z16)


def _tc_combine_body(s_ref, d_ref, r16_ref, b_ref, o_ref):
    s = s_ref[0] + s_ref[1]
    d = d_ref[0] + d_ref[1]
    r = 1.0 / (d + 1e-16)
    rex = jnp.dot(r, r16_ref[...], preferred_element_type=jnp.float32,
                  precision=lax.Precision.HIGHEST)
    o_ref[...] = jnp.maximum(s * rex + b_ref[...], 0.0)


def _tc_combine(S, D, R16, bias_row):
    grid = (N // _BN,)
    return pl.pallas_call(
        _tc_combine_body,
        grid=grid,
        in_specs=[
            pl.BlockSpec((2, _BN, HC), lambda i: (0, i, 0)),
            pl.BlockSpec((2, _BN, 16), lambda i: (0, i, 0)),
            pl.BlockSpec((16, HC), lambda i: (0, 0)),
            pl.BlockSpec((1, HC), lambda i: (0, 0)),
        ],
        out_specs=pl.BlockSpec((_BN, HC), lambda i: (i, 0)),
        out_shape=jax.ShapeDtypeStruct((N, HC), jnp.float32),
    )(S, D, R16, bias_row)


@jax.jit
def kernel(x, x_0, edge_index, W, att_src, att_dst, bias):
    del x_0

    loop = jnp.arange(N, dtype=jnp.int32)
    src = jnp.concatenate([edge_index[0].astype(jnp.int32), loop,
                           jnp.zeros((E_PAD - EP,), jnp.int32)])
    dst = jnp.concatenate([edge_index[1].astype(jnp.int32), loop,
                           jnp.full((E_PAD - EP,), N, jnp.int32)])

    lane = jnp.arange(HC, dtype=jnp.int32)
    head16 = jnp.arange(16, dtype=jnp.int32)
    Q = (head16[None, :] == (lane[:, None] // C)).astype(jnp.float32)
    R16 = ((head16[:, None] == (lane[None, :] // C)) &
           (head16[:, None] < H)).astype(jnp.float32)

    asf = att_src.reshape(1, HC)
    adf = att_dst.reshape(1, HC)
    pos = jnp.arange(HC, dtype=jnp.int32)
    perm = (2 * (pos // 32) + (pos % 2)) * C + (pos % 32) // 2
    W2 = W[:, perm]
    z128 = jnp.zeros((ROWS_PER_TILE, HC), jnp.float32)
    z16 = jnp.zeros((ROWS_PER_TILE, 16), jnp.float32)

    xpb, asrc, adst = _tc_prep(x, W, W2, asf, adf, Q)
    S, D = _sc_edge_pass(xpb, asrc, adst, src, dst, z128, z16)
    return _tc_combine(S, D, R16, bias.reshape(1, HC))


# speedup vs baseline: 1.1321x; 1.1321x over previous
import functools

import jax
import jax.numpy as jnp
from jax import lax
from jax.experimental import pallas as pl
from jax.experimental.pallas import tpu as pltpu
from jax.experimental.pallas import tpu_sc as plsc

N = 10000
IN = 128
H = 8
C = 16
HC = H * C
E = 320000
EP = E + N

K = 64
NW = 32
NB = 3
CH = 162
E_PAD = NW * K * CH
PER_W = CH * K

R_ROWS = 10240
ROWS_PER_TILE = R_ROWS // 16

_BN = 2000


def _tc_prep_body(x_ref, w_ref, asf_ref, adf_ref, q_ref, xp_ref, as_ref, ad_ref):
    xp = jnp.dot(x_ref[...], w_ref[...], preferred_element_type=jnp.float32,
                 precision=lax.Precision.HIGHEST)
    xp_ref[...] = xp
    as_ref[...] = jnp.dot(xp * asf_ref[...], q_ref[...],
                          preferred_element_type=jnp.float32,
                          precision=lax.Precision.HIGHEST)
    ad_ref[...] = jnp.dot(xp * adf_ref[...], q_ref[...],
                          preferred_element_type=jnp.float32,
                          precision=lax.Precision.HIGHEST)


def _tc_prep(x, W, att_src_flat, att_dst_flat, Q):
    grid = (N // _BN,)
    return pl.pallas_call(
        _tc_prep_body,
        grid=grid,
        in_specs=[
            pl.BlockSpec((_BN, IN), lambda i: (i, 0)),
            pl.BlockSpec((IN, HC), lambda i: (0, 0)),
            pl.BlockSpec((1, HC), lambda i: (0, 0)),
            pl.BlockSpec((1, HC), lambda i: (0, 0)),
            pl.BlockSpec((HC, 16), lambda i: (0, 0)),
        ],
        out_specs=[
            pl.BlockSpec((_BN, HC), lambda i: (i, 0)),
            pl.BlockSpec((_BN, 16), lambda i: (i, 0)),
            pl.BlockSpec((_BN, 16), lambda i: (i, 0)),
        ],
        out_shape=[
            jax.ShapeDtypeStruct((N, HC), jnp.float32),
            jax.ShapeDtypeStruct((N, 16), jnp.float32),
            jax.ShapeDtypeStruct((N, 16), jnp.float32),
        ],
    )(x, W, att_src_flat, att_dst_flat, Q)


def _splat(v, h):
    idx = jnp.full((16, 1), h, dtype=jnp.int32)
    dn = lax.GatherDimensionNumbers(offset_dims=(), collapsed_slice_dims=(0,),
                                    start_index_map=(0,))
    return lax.gather(v, idx, dn, slice_sizes=(1,),
                      mode=lax.GatherScatterMode.PROMISE_IN_BOUNDS)


def _sc_body(xp_h, as_h, ad_h, src_h, dst_h, z128_h, z16_h,
             s_out, d_out,
             s_sh, d_sh, srcv, dstv, dsts, rows, asv, adv, wv,
             isem_s, isem_d, gsem_r, gsem_a, gsem_b, ssem_s, ssem_d):
    cid = lax.axis_index("c")
    sid = lax.axis_index("s")
    wid = sid * 2 + cid

    pltpu.sync_copy(z128_h, s_sh.at[pl.ds(sid * ROWS_PER_TILE, ROWS_PER_TILE)])
    pltpu.sync_copy(z16_h, d_sh.at[pl.ds(sid * ROWS_PER_TILE, ROWS_PER_TILE)])
    plsc.subcore_barrier()

    base0 = wid * PER_W

    def idx_issue(ch, b):
        base = base0 + ch * K
        pltpu.async_copy(src_h.at[pl.ds(base, K)], srcv.at[b], isem_s.at[b])
        pltpu.async_copy(dst_h.at[pl.ds(base, K)], dstv.at[b], isem_d.at[b])

    def idx_wait(b):
        pltpu.make_async_copy(src_h.at[pl.ds(0, K)], srcv.at[b],
                              isem_s.at[b]).wait()
        pltpu.make_async_copy(dst_h.at[pl.ds(0, K)], dstv.at[b],
                              isem_d.at[b]).wait()

    def gather_issue(b):
        pltpu.async_copy(xp_h.at[srcv.at[b]], rows.at[b], gsem_r.at[b])
        pltpu.async_copy(as_h.at[srcv.at[b]], asv.at[b], gsem_a.at[b])
        pltpu.async_copy(ad_h.at[dstv.at[b]], adv.at[b], gsem_b.at[b])

    def gather_wait(b):
        pltpu.make_async_copy(xp_h.at[srcv.at[b]], rows.at[b],
                              gsem_r.at[b]).wait()
        pltpu.make_async_copy(as_h.at[srcv.at[b]], asv.at[b],
                              gsem_a.at[b]).wait()
        pltpu.make_async_copy(ad_h.at[dstv.at[b]], adv.at[b],
                              gsem_b.at[b]).wait()

    def scatter_issue(b):
        pltpu.async_copy(wv.at[b], d_sh.at[dsts.at[b]], ssem_d.at[b],
                         add=True)
        pltpu.async_copy(rows.at[b], s_sh.at[dsts.at[b]], ssem_s.at[b],
                         add=True)

    def scatter_wait(b):
        pltpu.make_async_copy(wv.at[b], d_sh.at[dsts.at[b]],
                              ssem_d.at[b]).wait()
        pltpu.make_async_copy(rows.at[b], s_sh.at[dsts.at[b]],
                              ssem_s.at[b]).wait()

    idx_issue(0, 0)
    idx_wait(0)
    gather_issue(0)
    idx_issue(1, 1)
    idx_wait(1)
    gather_issue(1)
    idx_issue(2, 2)

    @pl.loop(0, CH, step=NB)
    def _chunk(ch0):
        for b in range(NB):
            ch = ch0 + b
            nxt = (b + 2) % NB
            gather_wait(b)

            @pl.when(ch + 3 < CH)
            def _():
                idx_issue(ch + 3, b)

            @pl.when(ch + 2 < CH)
            def _():
                @pl.when(ch >= 1)
                def _():
                    scatter_wait(nxt)
                idx_wait(nxt)
                gather_issue(nxt)

            for j in range(K // 16):
                dsts.at[b][pl.ds(16 * j, 16)] = dstv.at[b][pl.ds(16 * j, 16)]

            @plsc.parallel_loop(0, K, unroll=4)
            def _edge(k):
                a = asv.at[b][k] + adv.at[b][k]
                w = jnp.exp(jnp.maximum(a, 0.2 * a))
                wv.at[b][k] = w
                for h in range(H):
                    sl = pl.ds(16 * h, 16)
                    rows.at[b][k, sl] = rows.at[b][k, sl] * _splat(w, h)

            scatter_issue(b)

    scatter_wait(0)
    scatter_wait(1)
    scatter_wait(2)

    plsc.subcore_barrier()
    sl = pl.ds(sid * ROWS_PER_TILE, ROWS_PER_TILE)
    pltpu.sync_copy(s_sh.at[sl], s_out.at[cid].at[sl])
    pltpu.sync_copy(d_sh.at[sl], d_out.at[cid].at[sl])


def _sc_edge_pass(xp, asrc, adst, srcp, dstp, z128, z16):
    mesh = plsc.VectorSubcoreMesh(core_axis_name="c", subcore_axis_name="s")
    f = pl.kernel(
        _sc_body,
        compiler_params=pltpu.CompilerParams(use_tc_tiling_on_sc=False),
        out_type=[
            jax.ShapeDtypeStruct((2, R_ROWS, HC), jnp.float32),
            jax.ShapeDtypeStruct((2, R_ROWS, 16), jnp.float32),
        ],
        mesh=mesh,
        scratch_types=[
            pltpu.VMEM_SHARED((R_ROWS, HC), jnp.float32),
            pltpu.VMEM_SHARED((R_ROWS, 16), jnp.float32),
            pltpu.VMEM((NB, K), jnp.int32),
            pltpu.VMEM((NB, K), jnp.int32),
            pltpu.VMEM((NB, K), jnp.int32),
            pltpu.VMEM((NB, K, HC), jnp.float32),
            pltpu.VMEM((NB, K, 16), jnp.float32),
            pltpu.VMEM((NB, K, 16), jnp.float32),
            pltpu.VMEM((NB, K, 16), jnp.float32),
            pltpu.SemaphoreType.DMA((NB,)),
            pltpu.SemaphoreType.DMA((NB,)),
            pltpu.SemaphoreType.DMA((NB,)),
            pltpu.SemaphoreType.DMA((NB,)),
            pltpu.SemaphoreType.DMA((NB,)),
            pltpu.SemaphoreType.DMA((NB,)),
            pltpu.SemaphoreType.DMA((NB,)),
        ],
    )
    return f(xp, asrc, adst, srcp, dstp, z128, z16)


def _tc_combine_body(s_ref, d_ref, r16_ref, b_ref, o_ref):
    s = s_ref[0] + s_ref[1]
    d = d_ref[0] + d_ref[1]
    r = 1.0 / (d + 1e-16)
    rex = jnp.dot(r, r16_ref[...], preferred_element_type=jnp.float32,
                  precision=lax.Precision.HIGHEST)
    o_ref[...] = jnp.maximum(s * rex + b_ref[...], 0.0)


def _tc_combine(S, D, R16, bias_row):
    grid = (N // _BN,)
    return pl.pallas_call(
        _tc_combine_body,
        grid=grid,
        in_specs=[
            pl.BlockSpec((2, _BN, HC), lambda i: (0, i, 0)),
            pl.BlockSpec((2, _BN, 16), lambda i: (0, i, 0)),
            pl.BlockSpec((16, HC), lambda i: (0, 0)),
            pl.BlockSpec((1, HC), lambda i: (0, 0)),
        ],
        out_specs=pl.BlockSpec((_BN, HC), lambda i: (i, 0)),
        out_shape=jax.ShapeDtypeStruct((N, HC), jnp.float32),
    )(S, D, R16, bias_row)


@jax.jit
def kernel(x, x_0, edge_index, W, att_src, att_dst, bias):
    del x_0

    loop = jnp.arange(N, dtype=jnp.int32)
    src = jnp.concatenate([edge_index[0].astype(jnp.int32), loop,
                           jnp.zeros((E_PAD - EP,), jnp.int32)])
    dst = jnp.concatenate([edge_index[1].astype(jnp.int32), loop,
                           jnp.full((E_PAD - EP,), N, jnp.int32)])

    lane = jnp.arange(HC, dtype=jnp.int32)
    head16 = jnp.arange(16, dtype=jnp.int32)
    Q = (head16[None, :] == (lane[:, None] // C)).astype(jnp.float32)
    R16 = ((head16[:, None] == (lane[None, :] // C)) &
           (head16[:, None] < H)).astype(jnp.float32)

    asf = att_src.reshape(1, HC)
    adf = att_dst.reshape(1, HC)
    z128 = jnp.zeros((ROWS_PER_TILE, HC), jnp.float32)
    z16 = jnp.zeros((ROWS_PER_TILE, 16), jnp.float32)

    xp, asrc, adst = _tc_prep(x, W, asf, adf, Q)
    S, D = _sc_edge_pass(xp, asrc, adst, src, dst, z128, z16)
    return _tc_combine(S, D, R16, bias.reshape(1, HC))


# speedup vs baseline: 1.1340x; 1.0016x over previous
import functools

import jax
import jax.numpy as jnp
from jax import lax
from jax.experimental import pallas as pl
from jax.experimental.pallas import tpu as pltpu
from jax.experimental.pallas import tpu_sc as plsc

N = 10000
IN = 128
H = 8
C = 16
HC = H * C
E = 320000
EP = E + N

K = 64
NW = 32
NB = 3
CH = 162
E_PAD = NW * K * CH
PER_W = CH * K

R_ROWS = 10240
ROWS_PER_TILE = R_ROWS // 16

_BN = 2000


def _tc_prep_body(x_ref, w_ref, asf_ref, adf_ref, q_ref, xp_ref, as_ref, ad_ref):
    xp = jnp.dot(x_ref[...], w_ref[...], preferred_element_type=jnp.float32,
                 precision=lax.Precision.HIGHEST)
    xp_ref[...] = xp
    as_ref[...] = jnp.dot(xp * asf_ref[...], q_ref[...],
                          preferred_element_type=jnp.float32,
                          precision=lax.Precision.HIGHEST)
    ad_ref[...] = jnp.dot(xp * adf_ref[...], q_ref[...],
                          preferred_element_type=jnp.float32,
                          precision=lax.Precision.HIGHEST)


def _tc_prep(x, W, att_src_flat, att_dst_flat, Q):
    grid = (N // _BN,)
    return pl.pallas_call(
        _tc_prep_body,
        grid=grid,
        in_specs=[
            pl.BlockSpec((_BN, IN), lambda i: (i, 0)),
            pl.BlockSpec((IN, HC), lambda i: (0, 0)),
            pl.BlockSpec((1, HC), lambda i: (0, 0)),
            pl.BlockSpec((1, HC), lambda i: (0, 0)),
            pl.BlockSpec((HC, 16), lambda i: (0, 0)),
        ],
        out_specs=[
            pl.BlockSpec((_BN, HC), lambda i: (i, 0)),
            pl.BlockSpec((_BN, 16), lambda i: (i, 0)),
            pl.BlockSpec((_BN, 16), lambda i: (i, 0)),
        ],
        out_shape=[
            jax.ShapeDtypeStruct((N, HC), jnp.float32),
            jax.ShapeDtypeStruct((N, 16), jnp.float32),
            jax.ShapeDtypeStruct((N, 16), jnp.float32),
        ],
    )(x, W, att_src_flat, att_dst_flat, Q)


def _splat(v, h):
    idx = jnp.full((16, 1), h, dtype=jnp.int32)
    dn = lax.GatherDimensionNumbers(offset_dims=(), collapsed_slice_dims=(0,),
                                    start_index_map=(0,))
    return lax.gather(v, idx, dn, slice_sizes=(1,),
                      mode=lax.GatherScatterMode.PROMISE_IN_BOUNDS)


def _sc_body(xp_h, as_h, ad_h, src_h, dst_h, z128_h, z16_h,
             s_out, d_out,
             s_sh, d_sh, srcv, dstv, dsts, rows, asv, adv, wv,
             isem_s, isem_d, gsem_r, gsem_a, gsem_b, ssem_s, ssem_d):
    cid = lax.axis_index("c")
    sid = lax.axis_index("s")
    wid = sid * 2 + cid

    pltpu.sync_copy(z128_h, s_sh.at[pl.ds(sid * ROWS_PER_TILE, ROWS_PER_TILE)])
    pltpu.sync_copy(z16_h, d_sh.at[pl.ds(sid * ROWS_PER_TILE, ROWS_PER_TILE)])
    plsc.subcore_barrier()

    base0 = wid * PER_W

    def idx_issue(ch, b):
        base = base0 + ch * K
        pltpu.async_copy(src_h.at[pl.ds(base, K)], srcv.at[b], isem_s.at[b])
        pltpu.async_copy(dst_h.at[pl.ds(base, K)], dstv.at[b], isem_d.at[b])

    def idx_wait(b):
        pltpu.make_async_copy(src_h.at[pl.ds(0, K)], srcv.at[b],
                              isem_s.at[b]).wait()
        pltpu.make_async_copy(dst_h.at[pl.ds(0, K)], dstv.at[b],
                              isem_d.at[b]).wait()

    def gather_issue(b):
        pltpu.async_copy(xp_h.at[srcv.at[b]], rows.at[b], gsem_r.at[b])
        pltpu.async_copy(as_h.at[srcv.at[b]], asv.at[b], gsem_a.at[b])
        pltpu.async_copy(ad_h.at[dstv.at[b]], adv.at[b], gsem_b.at[b])

    def gather_wait(b):
        pltpu.make_async_copy(xp_h.at[srcv.at[b]], rows.at[b],
                              gsem_r.at[b]).wait()
        pltpu.make_async_copy(as_h.at[srcv.at[b]], asv.at[b],
                              gsem_a.at[b]).wait()
        pltpu.make_async_copy(ad_h.at[dstv.at[b]], adv.at[b],
                              gsem_b.at[b]).wait()

    def scatter_issue(b):
        pltpu.async_copy(wv.at[b], d_sh.at[dsts.at[b]], ssem_d.at[b],
                         add=True)
        pltpu.async_copy(rows.at[b], s_sh.at[dsts.at[b]], ssem_s.at[b],
                         add=True)

    def scatter_wait(b):
        pltpu.make_async_copy(wv.at[b], d_sh.at[dsts.at[b]],
                              ssem_d.at[b]).wait()
        pltpu.make_async_copy(rows.at[b], s_sh.at[dsts.at[b]],
                              ssem_s.at[b]).wait()

    idx_issue(0, 0)
    idx_wait(0)
    gather_issue(0)
    idx_issue(1, 1)
    idx_wait(1)
    gather_issue(1)
    idx_issue(2, 2)

    @pl.loop(0, CH, step=NB)
    def _chunk(ch0):
        for b in range(NB):
            ch = ch0 + b
            nxt = (b + 2) % NB
            gather_wait(b)

            for j in range(K // 16):
                dsts.at[b][pl.ds(16 * j, 16)] = dstv.at[b][pl.ds(16 * j, 16)]

            @pl.when(ch + 3 < CH)
            def _():
                idx_issue(ch + 3, b)

            @pl.when(ch + 2 < CH)
            def _():
                @pl.when(ch >= 1)
                def _():
                    scatter_wait(nxt)
                idx_wait(nxt)
                gather_issue(nxt)

            @plsc.parallel_loop(0, K, unroll=4)
            def _edge(k):
                a = asv.at[b][k] + adv.at[b][k]
                w = jnp.exp(jnp.maximum(a, 0.2 * a))
                wv.at[b][k] = w
                for h in range(H):
                    sl = pl.ds(16 * h, 16)
                    rows.at[b][k, sl] = rows.at[b][k, sl] * _splat(w, h)

            scatter_issue(b)

    scatter_wait(0)
    scatter_wait(1)
    scatter_wait(2)

    plsc.subcore_barrier()
    sl = pl.ds(sid * ROWS_PER_TILE, ROWS_PER_TILE)
    pltpu.sync_copy(s_sh.at[sl], s_out.at[cid].at[sl])
    pltpu.sync_copy(d_sh.at[sl], d_out.at[cid].at[sl])


def _sc_edge_pass(xp, asrc, adst, srcp, dstp, z128, z16):
    mesh = plsc.VectorSubcoreMesh(core_axis_name="c", subcore_axis_name="s")
    f = pl.kernel(
        _sc_body,
        compiler_params=pltpu.CompilerParams(use_tc_tiling_on_sc=False),
        out_type=[
            jax.ShapeDtypeStruct((2, R_ROWS, HC), jnp.float32),
            jax.ShapeDtypeStruct((2, R_ROWS, 16), jnp.float32),
        ],
        mesh=mesh,
        scratch_types=[
            pltpu.VMEM_SHARED((R_ROWS, HC), jnp.float32),
            pltpu.VMEM_SHARED((R_ROWS, 16), jnp.float32),
            pltpu.VMEM((NB, K), jnp.int32),
            pltpu.VMEM((NB, K), jnp.int32),
            pltpu.VMEM((NB, K), jnp.int32),
            pltpu.VMEM((NB, K, HC), jnp.float32),
            pltpu.VMEM((NB, K, 16), jnp.float32),
            pltpu.VMEM((NB, K, 16), jnp.float32),
            pltpu.VMEM((NB, K, 16), jnp.float32),
            pltpu.SemaphoreType.DMA((NB,)),
            pltpu.SemaphoreType.DMA((NB,)),
            pltpu.SemaphoreType.DMA((NB,)),
            pltpu.SemaphoreType.DMA((NB,)),
            pltpu.SemaphoreType.DMA((NB,)),
            pltpu.SemaphoreType.DMA((NB,)),
            pltpu.SemaphoreType.DMA((NB,)),
        ],
    )
    return f(xp, asrc, adst, srcp, dstp, z128, z16)


def _tc_combine_body(s_ref, d_ref, r16_ref, b_ref, o_ref):
    s = s_ref[0] + s_ref[1]
    d = d_ref[0] + d_ref[1]
    r = 1.0 / (d + 1e-16)
    rex = jnp.dot(r, r16_ref[...], preferred_element_type=jnp.float32,
                  precision=lax.Precision.HIGHEST)
    o_ref[...] = jnp.maximum(s * rex + b_ref[...], 0.0)


def _tc_combine(S, D, R16, bias_row):
    grid = (N // _BN,)
    return pl.pallas_call(
        _tc_combine_body,
        grid=grid,
        in_specs=[
            pl.BlockSpec((2, _BN, HC), lambda i: (0, i, 0)),
            pl.BlockSpec((2, _BN, 16), lambda i: (0, i, 0)),
            pl.BlockSpec((16, HC), lambda i: (0, 0)),
            pl.BlockSpec((1, HC), lambda i: (0, 0)),
        ],
        out_specs=pl.BlockSpec((_BN, HC), lambda i: (i, 0)),
        out_shape=jax.ShapeDtypeStruct((N, HC), jnp.float32),
    )(S, D, R16, bias_row)


@jax.jit
def kernel(x, x_0, edge_index, W, att_src, att_dst, bias):
    del x_0

    loop = jnp.arange(N, dtype=jnp.int32)
    src = jnp.concatenate([edge_index[0].astype(jnp.int32), loop,
                           jnp.zeros((E_PAD - EP,), jnp.int32)])
    dst = jnp.concatenate([edge_index[1].astype(jnp.int32), loop,
                           jnp.full((E_PAD - EP,), N, jnp.int32)])

    lane = jnp.arange(HC, dtype=jnp.int32)
    head16 = jnp.arange(16, dtype=jnp.int32)
    Q = (head16[None, :] == (lane[:, None] // C)).astype(jnp.float32)
    R16 = ((head16[:, None] == (lane[None, :] // C)) &
           (head16[:, None] < H)).astype(jnp.float32)

    asf = att_src.reshape(1, HC)
    adf = att_dst.reshape(1, HC)
    z128 = jnp.zeros((ROWS_PER_TILE, HC), jnp.float32)
    z16 = jnp.zeros((ROWS_PER_TILE, 16), jnp.float32)

    xp, asrc, adst = _tc_prep(x, W, asf, adf, Q)
    S, D = _sc_edge_pass(xp, asrc, adst, src, dst, z128, z16)
    return _tc_combine(S, D, R16, bias.reshape(1, HC))


# speedup vs baseline: 1.4351x; 1.2656x over previous
import functools

import jax
import jax.numpy as jnp
from jax import lax
from jax.experimental import pallas as pl
from jax.experimental.pallas import tpu as pltpu
from jax.experimental.pallas import tpu_sc as plsc

N = 10000
IN = 128
H = 8
C = 16
HC = H * C
E = 320000
EP = E + N

K = 80
NW = 32
NB = 3
CH = 129
E_PAD = NW * K * CH
PER_W = CH * K

R_ROWS = 10016
ROWS_PER_TILE = R_ROWS // 16

_BN = 2000


def _tc_prep_body(x_ref, w_ref, asf_ref, adf_ref, q_ref, xp_ref, as_ref, ad_ref):
    xp = jnp.dot(x_ref[...], w_ref[...], preferred_element_type=jnp.float32,
                 precision=lax.Precision.HIGHEST)
    xp_ref[...] = xp
    as_ref[...] = jnp.dot(xp * asf_ref[...], q_ref[...],
                          preferred_element_type=jnp.float32,
                          precision=lax.Precision.HIGHEST)
    ad_ref[...] = jnp.dot(xp * adf_ref[...], q_ref[...],
                          preferred_element_type=jnp.float32,
                          precision=lax.Precision.HIGHEST)


def _tc_prep(x, W, att_src_flat, att_dst_flat, Q):
    grid = (N // _BN,)
    return pl.pallas_call(
        _tc_prep_body,
        grid=grid,
        in_specs=[
            pl.BlockSpec((_BN, IN), lambda i: (i, 0)),
            pl.BlockSpec((IN, HC), lambda i: (0, 0)),
            pl.BlockSpec((1, HC), lambda i: (0, 0)),
            pl.BlockSpec((1, HC), lambda i: (0, 0)),
            pl.BlockSpec((HC, 16), lambda i: (0, 0)),
        ],
        out_specs=[
            pl.BlockSpec((_BN, HC), lambda i: (i, 0)),
            pl.BlockSpec((_BN, 16), lambda i: (i, 0)),
            pl.BlockSpec((_BN, 16), lambda i: (i, 0)),
        ],
        out_shape=[
            jax.ShapeDtypeStruct((N, HC), jnp.float32),
            jax.ShapeDtypeStruct((N, 16), jnp.float32),
            jax.ShapeDtypeStruct((N, 16), jnp.float32),
        ],
    )(x, W, att_src_flat, att_dst_flat, Q)


def _splat(v, h):
    idx = jnp.full((16, 1), h, dtype=jnp.int32)
    dn = lax.GatherDimensionNumbers(offset_dims=(), collapsed_slice_dims=(0,),
                                    start_index_map=(0,))
    return lax.gather(v, idx, dn, slice_sizes=(1,),
                      mode=lax.GatherScatterMode.PROMISE_IN_BOUNDS)


def _sc_body(xp_h, as_h, ad_h, sidx_h, z128_h, z16_h,
             s_out, d_out,
             s_sh, d_sh, sidxv, dsts, rows, asv, adv,
             isem, gsem_r, gsem_a, gsem_b, ssem_s, ssem_d):
    cid = lax.axis_index("c")
    sid = lax.axis_index("s")
    wid = sid * 2 + cid

    pltpu.sync_copy(z128_h, s_sh.at[pl.ds(sid * ROWS_PER_TILE, ROWS_PER_TILE)])
    pltpu.sync_copy(z16_h, d_sh.at[pl.ds(sid * ROWS_PER_TILE, ROWS_PER_TILE)])
    plsc.subcore_barrier()

    base0 = wid * PER_W

    def idx_issue(ch, b):
        base = base0 + ch * K
        pltpu.async_copy(sidx_h.at[:, pl.ds(base, K)], sidxv.at[b],
                         isem.at[b])

    def idx_wait(b):
        pltpu.make_async_copy(sidx_h.at[:, pl.ds(0, K)], sidxv.at[b],
                              isem.at[b]).wait()

    def gather_issue(b):
        pltpu.async_copy(xp_h.at[sidxv.at[b, 0]], rows.at[b], gsem_r.at[b])
        pltpu.async_copy(as_h.at[sidxv.at[b, 0]], asv.at[b], gsem_a.at[b])
        pltpu.async_copy(ad_h.at[sidxv.at[b, 1]], adv.at[b], gsem_b.at[b])

    def gather_wait(b):
        pltpu.make_async_copy(xp_h.at[sidxv.at[b, 0]], rows.at[b],
                              gsem_r.at[b]).wait()
        pltpu.make_async_copy(as_h.at[sidxv.at[b, 0]], asv.at[b],
                              gsem_a.at[b]).wait()
        pltpu.make_async_copy(ad_h.at[sidxv.at[b, 1]], adv.at[b],
                              gsem_b.at[b]).wait()

    def scatter_issue(b):
        pltpu.async_copy(asv.at[b], d_sh.at[dsts.at[b]], ssem_d.at[b],
                         add=True)
        pltpu.async_copy(rows.at[b], s_sh.at[dsts.at[b]], ssem_s.at[b],
                         add=True)

    def scatter_wait(b):
        pltpu.make_async_copy(asv.at[b], d_sh.at[dsts.at[b]],
                              ssem_d.at[b]).wait()
        pltpu.make_async_copy(rows.at[b], s_sh.at[dsts.at[b]],
                              ssem_s.at[b]).wait()

    idx_issue(0, 0)
    idx_wait(0)
    gather_issue(0)
    idx_issue(1, 1)
    idx_wait(1)
    gather_issue(1)
    idx_issue(2, 2)

    @pl.loop(0, CH, step=NB)
    def _chunk(ch0):
        for b in range(NB):
            ch = ch0 + b
            nxt = (b + 2) % NB
            gather_wait(b)

            for j in range(K // 16):
                dsts.at[b][pl.ds(16 * j, 16)] = (
                    sidxv.at[b, 1][pl.ds(16 * j, 16)])

            @pl.when(ch + 3 < CH)
            def _():
                idx_issue(ch + 3, b)

            @pl.when(ch + 2 < CH)
            def _():
                @pl.when(ch >= 1)
                def _():
                    scatter_wait(nxt)
                idx_wait(nxt)
                gather_issue(nxt)

            @plsc.parallel_loop(0, K, unroll=4)
            def _edge(k):
                a = asv.at[b][k] + adv.at[b][k]
                w = jnp.exp(jnp.maximum(a, 0.2 * a))
                asv.at[b][k] = w
                for h in range(H):
                    sl = pl.ds(16 * h, 16)
                    rows.at[b][k, sl] = rows.at[b][k, sl] * _splat(w, h)

            scatter_issue(b)

    scatter_wait(0)
    scatter_wait(1)
    scatter_wait(2)

    plsc.subcore_barrier()
    sl = pl.ds(sid * ROWS_PER_TILE, ROWS_PER_TILE)
    pltpu.sync_copy(s_sh.at[sl], s_out.at[cid].at[sl])
    pltpu.sync_copy(d_sh.at[sl], d_out.at[cid].at[sl])


def _sc_edge_pass(xp, asrc, adst, sidx, z128, z16):
    mesh = plsc.VectorSubcoreMesh(core_axis_name="c", subcore_axis_name="s")
    f = pl.kernel(
        _sc_body,
        compiler_params=pltpu.CompilerParams(use_tc_tiling_on_sc=False),
        out_type=[
            jax.ShapeDtypeStruct((2, R_ROWS, HC), jnp.float32),
            jax.ShapeDtypeStruct((2, R_ROWS, 16), jnp.float32),
        ],
        mesh=mesh,
        scratch_types=[
            pltpu.VMEM_SHARED((R_ROWS, HC), jnp.float32),
            pltpu.VMEM_SHARED((R_ROWS, 16), jnp.float32),
            pltpu.VMEM((NB, 2, K), jnp.int32),
            pltpu.VMEM((NB, K), jnp.int32),
            pltpu.VMEM((NB, K, HC), jnp.float32),
            pltpu.VMEM((NB, K, 16), jnp.float32),
            pltpu.VMEM((NB, K, 16), jnp.float32),
            pltpu.SemaphoreType.DMA((NB,)),
            pltpu.SemaphoreType.DMA((NB,)),
            pltpu.SemaphoreType.DMA((NB,)),
            pltpu.SemaphoreType.DMA((NB,)),
            pltpu.SemaphoreType.DMA((NB,)),
            pltpu.SemaphoreType.DMA((NB,)),
        ],
    )
    return f(xp, asrc, adst, sidx, z128, z16)


def _tc_combine_body(s_ref, d_ref, r16_ref, b_ref, o_ref):
    s = s_ref[0] + s_ref[1]
    d = d_ref[0] + d_ref[1]
    r = 1.0 / (d + 1e-16)
    rex = jnp.dot(r, r16_ref[...], preferred_element_type=jnp.float32,
                  precision=lax.Precision.HIGHEST)
    o_ref[...] = jnp.maximum(s * rex + b_ref[...], 0.0)


def _tc_combine(S, D, R16, bias_row):
    grid = (N // _BN,)
    return pl.pallas_call(
        _tc_combine_body,
        grid=grid,
        in_specs=[
            pl.BlockSpec((2, _BN, HC), lambda i: (0, i, 0)),
            pl.BlockSpec((2, _BN, 16), lambda i: (0, i, 0)),
            pl.BlockSpec((16, HC), lambda i: (0, 0)),
            pl.BlockSpec((1, HC), lambda i: (0, 0)),
        ],
        out_specs=pl.BlockSpec((_BN, HC), lambda i: (i, 0)),
        out_shape=jax.ShapeDtypeStruct((N, HC), jnp.float32),
    )(S, D, R16, bias_row)


@jax.jit
def kernel(x, x_0, edge_index, W, att_src, att_dst, bias):
    del x_0

    loop = jnp.arange(N, dtype=jnp.int32)
    src = jnp.concatenate([edge_index[0].astype(jnp.int32), loop,
                           jnp.zeros((E_PAD - EP,), jnp.int32)])
    dst = jnp.concatenate([edge_index[1].astype(jnp.int32), loop,
                           jnp.full((E_PAD - EP,), N, jnp.int32)])
    sidx = jnp.stack([src, dst])

    lane = jnp.arange(HC, dtype=jnp.int32)
    head16 = jnp.arange(16, dtype=jnp.int32)
    Q = (head16[None, :] == (lane[:, None] // C)).astype(jnp.float32)
    R16 = ((head16[:, None] == (lane[None, :] // C)) &
           (head16[:, None] < H)).astype(jnp.float32)

    asf = att_src.reshape(1, HC)
    adf = att_dst.reshape(1, HC)
    z128 = jnp.zeros((ROWS_PER_TILE, HC), jnp.float32)
    z16 = jnp.zeros((ROWS_PER_TILE, 16), jnp.float32)

    xp, asrc, adst = _tc_prep(x, W, asf, adf, Q)
    S, D = _sc_edge_pass(xp, asrc, adst, sidx, z128, z16)
    return _tc_combine(S, D, R16, bias.reshape(1, HC))
